# baseline (device time: 19057 ns/iter reference)
import jax
import jax.numpy as jnp
from jax import lax
from jax.experimental import pallas as pl
from jax.experimental.pallas import tpu as pltpu

N_DEV = 8

_SCHEDULES = (
    (0, 384, (1, 3, 4)),
    (384, 384, (3, 4, 1)),
    (768, 256, (4, 1, 3)),
)
N_HALF = 2


def kernel(t):
    m, n = t.shape
    nc = n // N_HALF

    def body(x_ref, out_ref, *scratch):
        n_chains = len(_SCHEDULES) * N_HALF
        bufs = [scratch[7 * i:7 * i + 7] for i in range(n_chains)]
        send_sems, recv_sems = scratch[7 * n_chains], scratch[7 * n_chains + 1]

        my = lax.axis_index("i")
        k0 = lax.rem(my, 2)
        k1 = lax.rem(lax.div(my, 2), 2)
        k2 = lax.div(my, 4)
        partner = {
            1: my + (1 - 2 * k0),
            3: my + (1 - 2 * k0) + 2 * (1 - 2 * k1),
            4: my + 4 * (1 - 2 * k2),
        }
        bit = {1: lax.rem(k0 + k1, 2), 3: k1, 4: k2}

        barrier_sem = pltpu.get_barrier_semaphore()
        for mask in (1, 3, 4):
            pl.semaphore_signal(
                barrier_sem, inc=1,
                device_id=(partner[mask],),
                device_id_type=pl.DeviceIdType.MESH,
            )

        chains = []
        for h in range(N_HALF):
            for g, (roff, rows, masks) in enumerate(_SCHEDULES):
                h1 = rows // 2
                bA = bit[masks[0]]
                idx = h * len(_SCHEDULES) + g
                chains.append({
                    "masks": masks, "h1": h1,
                    "c0": h * nc,
                    "keep1": roff + bA * h1,
                    "send1": roff + (1 - bA) * h1,
                    "bufs": bufs[idx], "sem0": 4 * idx,
                })

        def start(src, dst, sem_idx, mask):
            rdma = pltpu.make_async_remote_copy(
                src_ref=src, dst_ref=dst,
                send_sem=send_sems.at[sem_idx],
                recv_sem=recv_sems.at[sem_idx],
                device_id=(partner[mask],),
                device_id_type=pl.DeviceIdType.MESH,
            )
            rdma.start()
            return rdma

        def cols(C):
            return slice(C["c0"], C["c0"] + nc)

        for C in chains:
            s1 = C["bufs"][0]
            s1[...] = x_ref[pl.ds(C["send1"], C["h1"]), cols(C)].astype(
                jnp.bfloat16)
        pl.semaphore_wait(barrier_sem, 3)
        for C in chains:
            C["rdma"] = start(C["bufs"][0], C["bufs"][1],
                              C["sem0"] + 0, C["masks"][0])


        for C in chains:
            C["rdma"].wait()
            acc = C["bufs"][6]
            acc[...] = (x_ref[pl.ds(C["keep1"], C["h1"]), cols(C)]
                        + C["bufs"][1][...].astype(jnp.float32))
            s2, r2 = C["bufs"][2], C["bufs"][3]
            s2[...] = acc[...].astype(jnp.bfloat16)
            C["rdma"] = start(s2, r2, C["sem0"] + 1, C["masks"][1])

        for C in chains:
            C["rdma"].wait()
            acc = C["bufs"][6]
            acc[...] += C["bufs"][3][...].astype(jnp.float32)
            s3, r3 = C["bufs"][4], C["bufs"][5]
            s3[...] = acc[...].astype(jnp.bfloat16)
            C["rdma"] = start(s3, r3, C["sem0"] + 2, C["masks"][2])

        for C in chains:
            C["rdma"].wait()
            acc = C["bufs"][6]
            s = acc[...] + C["bufs"][5][...].astype(jnp.float32)
            rr = jnp.maximum(s, 0.0)
            fv = jnp.tanh(s) * s * s + rr * rr * rr
            out_ref[pl.ds(C["keep1"], C["h1"]), cols(C)] = fv.astype(
                jnp.bfloat16)
            sl = out_ref.at[pl.ds(C["keep1"], C["h1"]), cols(C)]
            C["rdma"] = start(sl, sl, C["sem0"] + 3, C["masks"][0])

        for C in chains:
            C["rdma"].wait()

    comm_scratch = []
    n_chains = len(_SCHEDULES) * N_HALF
    for _ in range(N_HALF):
        for _, rows, _ in _SCHEDULES:
            h1 = rows // 2
            comm_scratch += [
                pltpu.VMEM((h1, nc), jnp.bfloat16),
                pltpu.VMEM((h1, nc), jnp.bfloat16),
                pltpu.VMEM((h1, nc), jnp.bfloat16),
                pltpu.VMEM((h1, nc), jnp.bfloat16),
                pltpu.VMEM((h1, nc), jnp.bfloat16),
                pltpu.VMEM((h1, nc), jnp.bfloat16),
                pltpu.VMEM((h1, nc), jnp.float32),
            ]

    return pl.pallas_call(
        body,
        out_shape=jax.ShapeDtypeStruct((m, n), jnp.bfloat16),
        in_specs=[pl.BlockSpec(memory_space=pltpu.VMEM)],
        out_specs=pl.BlockSpec(memory_space=pltpu.VMEM),
        scratch_shapes=[
            *comm_scratch,
            pltpu.SemaphoreType.DMA((4 * n_chains,)),
            pltpu.SemaphoreType.DMA((4 * n_chains,)),
        ],
        compiler_params=pltpu.CompilerParams(collective_id=0),
    )(t)


# device time: 17836 ns/iter; 1.0685x vs baseline; 1.0685x over previous
import jax
import jax.numpy as jnp
from jax import lax
from jax.experimental import pallas as pl
from jax.experimental.pallas import tpu as pltpu

N_DEV = 8

_SCHEDULES = (
    (0, 384, (1, 3, 4)),
    (384, 384, (3, 4, 1)),
    (768, 256, (4, 1, 3)),
)
N_HALF = 4


def kernel(t):
    m, n = t.shape
    nc = n // N_HALF

    def body(x_ref, out_ref, *scratch):
        n_chains = len(_SCHEDULES) * N_HALF
        bufs = [scratch[7 * i:7 * i + 7] for i in range(n_chains)]
        send_sems, recv_sems = scratch[7 * n_chains], scratch[7 * n_chains + 1]

        my = lax.axis_index("i")
        k0 = lax.rem(my, 2)
        k1 = lax.rem(lax.div(my, 2), 2)
        k2 = lax.div(my, 4)
        partner = {
            1: my + (1 - 2 * k0),
            3: my + (1 - 2 * k0) + 2 * (1 - 2 * k1),
            4: my + 4 * (1 - 2 * k2),
        }
        bit = {1: lax.rem(k0 + k1, 2), 3: k1, 4: k2}

        barrier_sem = pltpu.get_barrier_semaphore()
        for mask in (1, 3, 4):
            pl.semaphore_signal(
                barrier_sem, inc=1,
                device_id=(partner[mask],),
                device_id_type=pl.DeviceIdType.MESH,
            )

        chains = []
        for h in range(N_HALF):
            for g, (roff, rows, masks) in enumerate(_SCHEDULES):
                h1 = rows // 2
                bA = bit[masks[0]]
                idx = h * len(_SCHEDULES) + g
                chains.append({
                    "masks": masks, "h1": h1,
                    "c0": h * nc,
                    "keep1": roff + bA * h1,
                    "send1": roff + (1 - bA) * h1,
                    "bufs": bufs[idx], "sem0": 4 * idx,
                })

        def start(src, dst, sem_idx, mask):
            rdma = pltpu.make_async_remote_copy(
                src_ref=src, dst_ref=dst,
                send_sem=send_sems.at[sem_idx],
                recv_sem=recv_sems.at[sem_idx],
                device_id=(partner[mask],),
                device_id_type=pl.DeviceIdType.MESH,
            )
            rdma.start()
            return rdma

        def cols(C):
            return slice(C["c0"], C["c0"] + nc)

        for C in chains:
            s1 = C["bufs"][0]
            s1[...] = x_ref[pl.ds(C["send1"], C["h1"]), cols(C)].astype(
                jnp.bfloat16)
        pl.semaphore_wait(barrier_sem, 3)
        for C in chains:
            C["rdma"] = start(C["bufs"][0], C["bufs"][1],
                              C["sem0"] + 0, C["masks"][0])


        for C in chains:
            C["rdma"].wait()
            acc = C["bufs"][6]
            acc[...] = (x_ref[pl.ds(C["keep1"], C["h1"]), cols(C)]
                        + C["bufs"][1][...].astype(jnp.float32))
            s2, r2 = C["bufs"][2], C["bufs"][3]
            s2[...] = acc[...].astype(jnp.bfloat16)
            C["rdma"] = start(s2, r2, C["sem0"] + 1, C["masks"][1])

        for C in chains:
            C["rdma"].wait()
            acc = C["bufs"][6]
            acc[...] += C["bufs"][3][...].astype(jnp.float32)
            s3, r3 = C["bufs"][4], C["bufs"][5]
            s3[...] = acc[...].astype(jnp.bfloat16)
            C["rdma"] = start(s3, r3, C["sem0"] + 2, C["masks"][2])

        for C in chains:
            C["rdma"].wait()
            acc = C["bufs"][6]
            s = acc[...] + C["bufs"][5][...].astype(jnp.float32)
            rr = jnp.maximum(s, 0.0)
            fv = jnp.tanh(s) * s * s + rr * rr * rr
            out_ref[pl.ds(C["keep1"], C["h1"]), cols(C)] = fv.astype(
                jnp.bfloat16)
            sl = out_ref.at[pl.ds(C["keep1"], C["h1"]), cols(C)]
            C["rdma"] = start(sl, sl, C["sem0"] + 3, C["masks"][0])

        for C in chains:
            C["rdma"].wait()

    comm_scratch = []
    n_chains = len(_SCHEDULES) * N_HALF
    for _ in range(N_HALF):
        for _, rows, _ in _SCHEDULES:
            h1 = rows // 2
            comm_scratch += [
                pltpu.VMEM((h1, nc), jnp.bfloat16),
                pltpu.VMEM((h1, nc), jnp.bfloat16),
                pltpu.VMEM((h1, nc), jnp.bfloat16),
                pltpu.VMEM((h1, nc), jnp.bfloat16),
                pltpu.VMEM((h1, nc), jnp.bfloat16),
                pltpu.VMEM((h1, nc), jnp.bfloat16),
                pltpu.VMEM((h1, nc), jnp.float32),
            ]

    return pl.pallas_call(
        body,
        out_shape=jax.ShapeDtypeStruct((m, n), jnp.bfloat16),
        in_specs=[pl.BlockSpec(memory_space=pltpu.VMEM)],
        out_specs=pl.BlockSpec(memory_space=pltpu.VMEM),
        scratch_shapes=[
            *comm_scratch,
            pltpu.SemaphoreType.DMA((4 * n_chains,)),
            pltpu.SemaphoreType.DMA((4 * n_chains,)),
        ],
        compiler_params=pltpu.CompilerParams(collective_id=0),
    )(t)
